# baseline (speedup 1.0000x reference)
import jax
import jax.numpy as jnp
from jax import lax
from jax.experimental import pallas as pl
from jax.experimental.pallas import tpu as pltpu
from jax.experimental.pallas import tpu_sc as plsc

M = 2048
C = 128
NC, NS = 2, 16
MP = M + 16


def _sc_segsum(Uold, gidx, sidx, n_chunks, n, d):
    def body(u_hbm, gidx_hbm, sidx_hbm, out_hbm,
             tab, acc, ig, isA, gba, gbb, zrow, semA, semB):
        c = lax.axis_index("c")
        s = lax.axis_index("s")

        z = jnp.zeros((16,), jnp.float32)
        for i in range(16):
            for k in range(d // 16):
                zrow[i, pl.ds(k * 16, 16)] = z
        rows_per_tile = M // NS

        pltpu.sync_copy(gidx_hbm.at[s], ig)
        pltpu.sync_copy(sidx_hbm.at[s], isA)

        tab_rows_per_tile = n // NS

        for j in range(2):
            for r in range(rows_per_tile // 16):
                pltpu.sync_copy(zrow, acc.at[pl.ds(s * rows_per_tile + r * 16, 16)])

            @pl.when(s == NS - 1)
            def _zero_dummy():
                pltpu.sync_copy(zrow, acc.at[pl.ds(M, 16)])

            for r in range(tab_rows_per_tile // C):
                row0 = s * tab_rows_per_tile + r * C
                pltpu.sync_copy(u_hbm.at[2 * c + j, pl.ds(row0, C)], gba)
                pltpu.sync_copy(gba, tab.at[pl.ds(row0, C)])

            plsc.subcore_barrier()

            def start_spmem_gather(g, gb, sem):
                pltpu.async_copy(tab.at[ig.at[g]], gb, sem)

            def start_hbm_gather(g, gb, sem):
                pltpu.async_copy(u_hbm.at[2 * c + j].at[ig.at[g]], gb, sem)

            def finish_spmem_chunk(g, gb, sem):
                pltpu.make_async_copy(tab.at[ig.at[g]], gb, sem).wait()
                pltpu.sync_copy(gb, acc.at[isA.at[g]], add=True)

            def finish_hbm_chunk(g, gb, sem):
                pltpu.make_async_copy(u_hbm.at[2 * c + j].at[ig.at[g]], gb, sem).wait()
                pltpu.sync_copy(gb, acc.at[isA.at[g]], add=True)

            start_spmem_gather(0, gba, semA)

            def outer(t, carry):
                g0 = 2 * t
                start_hbm_gather(g0 + 1, gbb, semB)
                finish_spmem_chunk(g0, gba, semA)

                @pl.when(g0 + 2 < n_chunks)
                def _prefetch():
                    start_spmem_gather(g0 + 2, gba, semA)

                finish_hbm_chunk(g0 + 1, gbb, semB)
                return carry

            lax.fori_loop(0, n_chunks // 2, outer, 0)

            plsc.subcore_barrier()

            pltpu.sync_copy(acc.at[pl.ds(s * rows_per_tile, rows_per_tile)], gbb)
            pltpu.sync_copy(gbb, out_hbm.at[c, j, pl.ds(s * rows_per_tile, rows_per_tile)])

    fn = pl.kernel(
        body,
        out_type=jax.ShapeDtypeStruct((NC, 2, M, d), jnp.float32),
        mesh=plsc.VectorSubcoreMesh(core_axis_name="c", subcore_axis_name="s"),
        scratch_types=[
            pltpu.VMEM_SHARED((n, d), jnp.float32),
            pltpu.VMEM_SHARED((MP, d), jnp.float32),
            pltpu.VMEM((n_chunks, C), jnp.int32),
            pltpu.VMEM((n_chunks, C), jnp.int32),
            pltpu.VMEM((C, d), jnp.float32),
            pltpu.VMEM((C, d), jnp.float32),
            pltpu.VMEM((16, d), jnp.float32),
            pltpu.SemaphoreType.DMA,
            pltpu.SemaphoreType.DMA,
        ],
    )
    return fn(Uold, gidx, sidx)


def kernel(Uold, src, dst):
    b, n, d = Uold.shape
    e = src.shape[0]

    gran = 2 * NS * C
    e_pad = ((e + gran - 1) // gran) * gran
    n_chunks = e_pad // (NS * C)
    pad = e_pad - e
    src_p = jnp.concatenate([src, jnp.zeros((pad,), jnp.int32)])
    dst_p = jnp.concatenate([dst, jnp.full((pad,), M, jnp.int32)])

    gidx = src_p.reshape(NS, n_chunks, C)
    sidx = dst_p.reshape(NS, n_chunks, C)

    out4 = _sc_segsum(Uold, gidx, sidx, n_chunks, n, d)
    return out4.reshape(b, M, d)
